# baseline (speedup 1.0000x reference)
import functools

import jax
import jax.numpy as jnp
from jax import lax
from jax.experimental import pallas as pl
from jax.experimental.pallas import tpu as pltpu
from jax.experimental.pallas import tpu_sc as plsc

_L = 16
_WIN = 512
_WSH = 9
_IDXCH = 1024
_CAP = 6144
_TPAD = 128


def _iota16():
    return lax.iota(jnp.int32, _L)


def _make_stage_kernel(B, V, D, n_workers, num_cores):
    n_win_full = V // _WIN
    tail = V - n_win_full * _WIN
    n_win = n_win_full + (1 if tail else 0)
    j_max = -(-n_win // n_workers)
    tri_trips = -(-j_max // 3)
    n_rounds = -(-B // _CAP)
    mesh = plsc.VectorSubcoreMesh(core_axis_name="c", subcore_axis_name="s")

    @functools.partial(
        pl.kernel,
        out_type=(jax.ShapeDtypeStruct((B + n_workers * _L, 2 * D), jnp.float32),
                  jax.ShapeDtypeStruct((B + n_workers * _L, 2 * D), jnp.float32)),
        mesh=mesh,
        scratch_types=[
            pltpu.VMEM((2, _IDXCH), jnp.int32),
            pltpu.VMEM((_CAP,), jnp.int32),
            pltpu.VMEM((_CAP,), jnp.int32),
            pltpu.VMEM((_CAP,), jnp.int32),
            pltpu.VMEM((_CAP,), jnp.int32),
            pltpu.VMEM((3, D, _WIN), jnp.float32),
            pltpu.VMEM((2, _L, 2 * D), jnp.float32),
            pltpu.SemaphoreType.DMA,
            pltpu.SemaphoreType.DMA,
            pltpu.SemaphoreType.DMA,
            pltpu.SemaphoreType.DMA,
            pltpu.SemaphoreType.DMA,
        ],
        compiler_params=pltpu.CompilerParams(needs_layout_passes=False),
    )
    def stage_kernel(center_hbm, context_hbm, int_hbm, outt_hbm,
                     intt_hbm, outtt_hbm,
                     stc_hbm, stx_hbm,
                     idxbuf, wl_v, wl_p, ml_v, ml_p, slab, tok,
                     ssem0, ssem1, ssem2, csem0, csem1):
        wid = lax.axis_index("s") * num_cores + lax.axis_index("c")
        ssems = (ssem0, ssem1, ssem2)
        csems = (csem0, csem1)

        def run_phase(idx_hbm, tbl_hbm, tail_hbm, staged_hbm):

            def build(r):
                nch = B // _IDXCH

                def fire_idx(ch, slot):
                    @pl.when(ch < nch)
                    def _():
                        pltpu.async_copy(
                            idx_hbm.at[pl.ds(ch * _IDXCH, _IDXCH)],
                            idxbuf.at[slot], ssems[slot])

                def wait_idx(ch, slot):
                    @pl.when(ch < nch)
                    def _():
                        pltpu.make_async_copy(
                            idx_hbm.at[pl.ds(0, _IDXCH)],
                            idxbuf.at[slot], ssems[slot]).wait()

                def scan_chunk(ch, slot, cnt):
                    def scan16(s, cnt):
                        v = idxbuf[slot, pl.ds(s * _L, _L)]
                        m = (lax.shift_right_logical(v, _WSH) % n_workers
                             == wid)
                        mi = m.astype(jnp.int32)
                        ordin = cnt + plsc.cumsum(mi) - mi
                        m2 = m & (ordin >= r * _CAP) & (ordin < (r + 1) * _CAP)
                        pos = ch * _IDXCH + s * _L + _iota16()
                        off = jnp.clip(cnt - r * _CAP, 0, _CAP)
                        plsc.store_compressed(wl_v.at[pl.ds(off, _L)], v,
                                              mask=m2)
                        plsc.store_compressed(wl_p.at[pl.ds(off, _L)], pos,
                                              mask=m2)
                        return cnt + jnp.sum(mi)

                    return lax.fori_loop(0, _IDXCH // _L, scan16, cnt)

                fire_idx(0, 0)

                def bpair(tt, cnt):
                    ch0 = 2 * tt
                    fire_idx(ch0 + 1, 1)
                    wait_idx(ch0, 0)
                    cnt = scan_chunk(ch0, 0, cnt)
                    fire_idx(ch0 + 2, 0)
                    wait_idx(ch0 + 1, 1)
                    cnt = scan_chunk(ch0 + 1, 1, cnt)
                    return cnt

                return lax.fori_loop(0, (nch + 1) // 2, bpair, 0)

            def windows(rcnt):
                def fire(j, slot):
                    sid = wid + n_workers * j

                    @pl.when(sid < n_win_full)
                    def _():
                        pltpu.async_copy(
                            tbl_hbm.at[:, pl.ds(sid * _WIN, _WIN)],
                            slab.at[slot], ssems[slot])

                    if tail:
                        @pl.when(sid == n_win_full)
                        def _():
                            pltpu.async_copy(
                                tail_hbm,
                                slab.at[slot].at[:, pl.ds(0, _TPAD)],
                                ssems[slot])

                def drain_slab(j, slot):
                    sid = wid + n_workers * j

                    @pl.when(sid < n_win_full)
                    def _():
                        pltpu.make_async_copy(
                            tbl_hbm.at[:, pl.ds(0, _WIN)],
                            slab.at[slot], ssems[slot]).wait()

                    if tail:
                        @pl.when(sid == n_win_full)
                        def _():
                            pltpu.make_async_copy(
                                tail_hbm,
                                slab.at[slot].at[:, pl.ds(0, _TPAD)],
                                ssems[slot]).wait()

                def extract_group(mlv, mlp, src_ref, base, mcnt, tslot):
                    vv = mlv[pl.ds(base, _L)]
                    pp = mlp[pl.ds(base, _L)]
                    lm = _iota16() < (mcnt - base)
                    vloc = jnp.where(lm, vv % _WIN, 0)
                    posc = jnp.where(lm, pp, B + wid * _L + _iota16())
                    for d in range(D):
                        g = plsc.load_gather(
                            src_ref, [jnp.full((_L,), d, jnp.int32), vloc])
                        plsc.store_scatter(
                            tok.at[tslot],
                            [_iota16(), jnp.full((_L,), d, jnp.int32)], g)
                    pltpu.async_copy(tok.at[tslot], staged_hbm.at[posc],
                                     csems[tslot])

                def drain_scatter(tslot):
                    pltpu.make_async_copy(tok.at[tslot],
                                          staged_hbm.at[pl.ds(0, _L)],
                                          csems[tslot]).wait()

                def scan1(sid):
                    def scan_wl(i, mc):
                        v = wl_v[pl.ds(i * _L, _L)]
                        p = wl_p[pl.ds(i * _L, _L)]
                        valid = _iota16() < (rcnt - i * _L)
                        m = valid & (lax.shift_right_logical(v, _WSH) == sid)
                        plsc.store_compressed(ml_v.at[pl.ds(mc, _L)], v,
                                              mask=m)
                        plsc.store_compressed(ml_p.at[pl.ds(mc, _L)], p,
                                              mask=m)
                        return mc + jnp.sum(m.astype(jnp.int32))

                    return lax.fori_loop(0, (rcnt + _L - 1) // _L, scan_wl, 0)

                def groups(mlv, mlp, mcnt, src_ref):
                    n_g = (mcnt + _L - 1) // _L

                    def gpair(k, carry):
                        @pl.when(k > 0)
                        def _():
                            drain_scatter(0)
                            drain_scatter(1)

                        extract_group(mlv, mlp, src_ref, 2 * k * _L, mcnt, 0)

                        @pl.when(2 * k + 1 < n_g)
                        def _():
                            extract_group(mlv, mlp, src_ref,
                                          (2 * k + 1) * _L, mcnt, 1)
                        return carry

                    lax.fori_loop(0, (n_g + 1) // 2, gpair, 0)
                    rem = n_g - 2 * ((n_g + 1) // 2 - 1)

                    @pl.when(n_g > 0)
                    def _():
                        drain_scatter(0)

                        @pl.when(rem >= 2)
                        def _():
                            drain_scatter(1)

                fire(0, 0)
                fire(1, 1)

                def one(j, slot, next_j, next_slot):
                    sid = wid + n_workers * j
                    mc = scan1(sid)
                    fire(next_j, next_slot)
                    drain_slab(j, slot)

                    @pl.when(sid < n_win)
                    def _():
                        groups(ml_v, ml_p, mc, slab.at[slot])

                def wtri(t, carry):
                    j0 = 3 * t
                    one(j0, 0, j0 + 2, 2)
                    one(j0 + 1, 1, j0 + 3, 0)
                    one(j0 + 2, 2, j0 + 4, 1)
                    return carry

                lax.fori_loop(0, tri_trips, wtri, 0)

            total = build(0)
            windows(jnp.minimum(total, _CAP))
            if n_rounds > 1:
                def extra_round(r, tot):
                    @pl.when(tot > r * _CAP)
                    def _():
                        build(r)
                        windows(jnp.clip(tot - r * _CAP, 0, _CAP))
                    return tot

                lax.fori_loop(1, n_rounds, extra_round, total)

        run_phase(center_hbm, int_hbm, intt_hbm, stc_hbm)
        run_phase(context_hbm, outt_hbm, outtt_hbm, stx_hbm)

    return stage_kernel


def _dot_tc(c_ref, x_ref, o_ref):
    D = c_ref.shape[1] // 2
    prod = c_ref[:, :D] * x_ref[:, :D]
    o_ref[...] = jnp.sum(prod, axis=1)


def kernel(center, context, in_table, out_table):
    B, = center.shape
    V, D = in_table.shape
    info = plsc.get_sparse_core_info()
    n_workers = info.num_cores * info.num_subcores
    stage = _make_stage_kernel(B, V, D, n_workers, info.num_cores)
    tail = V % _WIN
    v0 = V - tail if tail else V - 1
    pad = _TPAD - (V - v0)
    in_tail = jnp.pad(in_table[v0:, :].T, ((0, 0), (0, pad)))
    out_tail = jnp.pad(out_table[v0:, :].T, ((0, 0), (0, pad)))
    stc, stx = stage(center, context, in_table.T, out_table.T,
                     in_tail, out_tail)

    blk = 2048
    dot = pl.pallas_call(
        _dot_tc,
        grid=(B // blk,),
        in_specs=[
            pl.BlockSpec((blk, 2 * D), lambda i: (i, 0)),
            pl.BlockSpec((blk, 2 * D), lambda i: (i, 0)),
        ],
        out_specs=pl.BlockSpec((blk,), lambda i: (i,)),
        out_shape=jax.ShapeDtypeStruct((B,), jnp.float32),
    )
    return dot(stc[:B], stx[:B])


# speedup vs baseline: 1.0392x; 1.0392x over previous
import functools

import jax
import jax.numpy as jnp
from jax import lax
from jax.experimental import pallas as pl
from jax.experimental.pallas import tpu as pltpu
from jax.experimental.pallas import tpu_sc as plsc

_L = 16
_WIN = 512
_WSH = 9
_IDXCH = 1024
_CAP = 6144
_TPAD = 128


def _iota16():
    return lax.iota(jnp.int32, _L)


def _make_stage_kernel(B, V, D, n_workers, num_cores):
    n_win_full = V // _WIN
    tail = V - n_win_full * _WIN
    n_win = n_win_full + (1 if tail else 0)
    j_max = -(-n_win // n_workers)
    tri_trips = -(-j_max // 3)
    n_rounds = -(-B // _CAP)
    mesh = plsc.VectorSubcoreMesh(core_axis_name="c", subcore_axis_name="s")

    @functools.partial(
        pl.kernel,
        out_type=(jax.ShapeDtypeStruct((B + n_workers * _L, 2 * D), jnp.float32),
                  jax.ShapeDtypeStruct((B + n_workers * _L, 2 * D), jnp.float32)),
        mesh=mesh,
        scratch_types=[
            pltpu.VMEM((2, _IDXCH), jnp.int32),
            pltpu.VMEM((_CAP,), jnp.int32),
            pltpu.VMEM((_CAP,), jnp.int32),
            pltpu.VMEM((_CAP,), jnp.int32),
            pltpu.VMEM((_CAP,), jnp.int32),
            pltpu.VMEM((3, D, _WIN), jnp.float32),
            pltpu.VMEM((2, _L, 2 * D), jnp.float32),
            pltpu.SemaphoreType.DMA,
            pltpu.SemaphoreType.DMA,
            pltpu.SemaphoreType.DMA,
            pltpu.SemaphoreType.DMA,
            pltpu.SemaphoreType.DMA,
        ],
        compiler_params=pltpu.CompilerParams(needs_layout_passes=False),
    )
    def stage_kernel(center_hbm, context_hbm, int_hbm, outt_hbm,
                     intt_hbm, outtt_hbm,
                     stc_hbm, stx_hbm,
                     idxbuf, wl_v, wl_p, ml_v, ml_p, slab, tok,
                     ssem0, ssem1, ssem2, csem0, csem1):
        wid = lax.axis_index("s") * num_cores + lax.axis_index("c")
        ssems = (ssem0, ssem1, ssem2)
        csems = (csem0, csem1)

        def run_phase(idx_hbm, tbl_hbm, tail_hbm, staged_hbm):

            def build(r):
                nch = B // _IDXCH

                def fire_idx(ch, slot):
                    @pl.when(ch < nch)
                    def _():
                        pltpu.async_copy(
                            idx_hbm.at[pl.ds(ch * _IDXCH, _IDXCH)],
                            idxbuf.at[slot], ssems[slot])

                def wait_idx(ch, slot):
                    @pl.when(ch < nch)
                    def _():
                        pltpu.make_async_copy(
                            idx_hbm.at[pl.ds(0, _IDXCH)],
                            idxbuf.at[slot], ssems[slot]).wait()

                def scan_chunk(ch, slot, cnt):
                    def scan16(s, cnt):
                        v = idxbuf[slot, pl.ds(s * _L, _L)]
                        m = (lax.shift_right_logical(v, _WSH) % n_workers
                             == wid)
                        mi = m.astype(jnp.int32)
                        ordin = cnt + plsc.cumsum(mi) - mi
                        m2 = m & (ordin >= r * _CAP) & (ordin < (r + 1) * _CAP)
                        pos = ch * _IDXCH + s * _L + _iota16()
                        off = jnp.clip(cnt - r * _CAP, 0, _CAP)
                        plsc.store_compressed(wl_v.at[pl.ds(off, _L)], v,
                                              mask=m2)
                        plsc.store_compressed(wl_p.at[pl.ds(off, _L)], pos,
                                              mask=m2)
                        return cnt + jnp.sum(mi)

                    return lax.fori_loop(0, _IDXCH // _L, scan16, cnt)

                fire_idx(0, 0)

                def bpair(tt, cnt):
                    ch0 = 2 * tt
                    fire_idx(ch0 + 1, 1)
                    wait_idx(ch0, 0)
                    cnt = scan_chunk(ch0, 0, cnt)
                    fire_idx(ch0 + 2, 0)
                    wait_idx(ch0 + 1, 1)
                    cnt = scan_chunk(ch0 + 1, 1, cnt)
                    return cnt

                return lax.fori_loop(0, (nch + 1) // 2, bpair, 0)

            def windows(rcnt):
                def fire(j, slot):
                    sid = wid + n_workers * j

                    @pl.when(sid < n_win_full)
                    def _():
                        pltpu.async_copy(
                            tbl_hbm.at[:, pl.ds(sid * _WIN, _WIN)],
                            slab.at[slot], ssems[slot])

                    if tail:
                        @pl.when(sid == n_win_full)
                        def _():
                            pltpu.async_copy(
                                tail_hbm,
                                slab.at[slot].at[:, pl.ds(0, _TPAD)],
                                ssems[slot])

                def drain_slab(j, slot):
                    sid = wid + n_workers * j

                    @pl.when(sid < n_win_full)
                    def _():
                        pltpu.make_async_copy(
                            tbl_hbm.at[:, pl.ds(0, _WIN)],
                            slab.at[slot], ssems[slot]).wait()

                    if tail:
                        @pl.when(sid == n_win_full)
                        def _():
                            pltpu.make_async_copy(
                                tail_hbm,
                                slab.at[slot].at[:, pl.ds(0, _TPAD)],
                                ssems[slot]).wait()

                def extract_group(mlv, mlp, src_ref, base, mcnt, tslot):
                    vv = mlv[pl.ds(base, _L)]
                    pp = mlp[pl.ds(base, _L)]
                    lm = _iota16() < (mcnt - base)
                    vloc = jnp.where(lm, vv % _WIN, 0)
                    posc = jnp.where(lm, pp, B + wid * _L + _iota16())
                    for d in range(D):
                        g = plsc.load_gather(
                            src_ref, [jnp.full((_L,), d, jnp.int32), vloc])
                        plsc.store_scatter(
                            tok.at[tslot],
                            [_iota16(), jnp.full((_L,), d, jnp.int32)], g)
                    pltpu.async_copy(tok.at[tslot], staged_hbm.at[posc],
                                     csems[tslot])

                def drain_scatter(tslot):
                    pltpu.make_async_copy(tok.at[tslot],
                                          staged_hbm.at[pl.ds(0, _L)],
                                          csems[tslot]).wait()

                def scan1(sid):
                    def scan_wl(i, mc):
                        v = wl_v[pl.ds(i * _L, _L)]
                        p = wl_p[pl.ds(i * _L, _L)]
                        valid = _iota16() < (rcnt - i * _L)
                        m = valid & (lax.shift_right_logical(v, _WSH) == sid)
                        plsc.store_compressed(ml_v.at[pl.ds(mc, _L)], v,
                                              mask=m)
                        plsc.store_compressed(ml_p.at[pl.ds(mc, _L)], p,
                                              mask=m)
                        return mc + jnp.sum(m.astype(jnp.int32))

                    return lax.fori_loop(0, (rcnt + _L - 1) // _L, scan_wl, 0)

                def groups(mlv, mlp, mcnt, src_ref):
                    n_g = (mcnt + _L - 1) // _L

                    def gpair(k, carry):
                        @pl.when(k > 0)
                        def _():
                            drain_scatter(0)
                            drain_scatter(1)

                        extract_group(mlv, mlp, src_ref, 2 * k * _L, mcnt, 0)

                        @pl.when(2 * k + 1 < n_g)
                        def _():
                            extract_group(mlv, mlp, src_ref,
                                          (2 * k + 1) * _L, mcnt, 1)
                        return carry

                    lax.fori_loop(0, (n_g + 1) // 2, gpair, 0)
                    rem = n_g - 2 * ((n_g + 1) // 2 - 1)

                    @pl.when(n_g > 0)
                    def _():
                        drain_scatter(0)

                        @pl.when(rem >= 2)
                        def _():
                            drain_scatter(1)

                fire(0, 0)
                fire(1, 1)

                def one(j, slot, next_j, next_slot):
                    sid = wid + n_workers * j
                    mc = scan1(sid)
                    fire(next_j, next_slot)
                    drain_slab(j, slot)

                    @pl.when(sid < n_win)
                    def _():
                        groups(ml_v, ml_p, mc, slab.at[slot])

                def wtri(t, carry):
                    j0 = 3 * t
                    one(j0, 0, j0 + 2, 2)
                    one(j0 + 1, 1, j0 + 3, 0)
                    one(j0 + 2, 2, j0 + 4, 1)
                    return carry

                lax.fori_loop(0, tri_trips, wtri, 0)

            total = build(0)
            windows(jnp.minimum(total, _CAP))
            if n_rounds > 1:
                def extra_round(r, tot):
                    @pl.when(tot > r * _CAP)
                    def _():
                        build(r)
                        windows(jnp.clip(tot - r * _CAP, 0, _CAP))
                    return tot

                lax.fori_loop(1, n_rounds, extra_round, total)

        run_phase(center_hbm, int_hbm, intt_hbm, stc_hbm)
        run_phase(context_hbm, outt_hbm, outtt_hbm, stx_hbm)

    return stage_kernel


def _dot_tc(c_ref, x_ref, o_ref):
    D = c_ref.shape[1] // 2
    B = o_ref.shape[0]
    prod = c_ref[:B, :D] * x_ref[:B, :D]
    o_ref[...] = jnp.sum(prod, axis=1)


def kernel(center, context, in_table, out_table):
    B, = center.shape
    V, D = in_table.shape
    info = plsc.get_sparse_core_info()
    n_workers = info.num_cores * info.num_subcores
    stage = _make_stage_kernel(B, V, D, n_workers, info.num_cores)
    tail = V % _WIN
    v0 = V - tail if tail else V - 1
    pad = _TPAD - (V - v0)
    in_tail = jnp.pad(in_table[v0:, :].T, ((0, 0), (0, pad)))
    out_tail = jnp.pad(out_table[v0:, :].T, ((0, 0), (0, pad)))
    stc, stx = stage(center, context, in_table.T, out_table.T,
                     in_tail, out_tail)

    dot = pl.pallas_call(
        _dot_tc,
        out_shape=jax.ShapeDtypeStruct((B,), jnp.float32),
    )
    return dot(stc, stx)
